# baseline (device time: 63345 ns/iter reference)
import jax
import jax.numpy as jnp
from jax import lax
from jax.experimental import pallas as pl
from jax.experimental.pallas import tpu as pltpu

N_DEV = 8
B_LOC = 2
H_LOC = 4
SQ = 128
DH = 64
D_MODEL = 512
D_CHUNK = H_LOC * DH

_SLOT_ORDER = (1, 7, 2, 6, 3, 5, 4)


def kernel(x, Wq, K_ext, V_ext, Wo):
    my = lax.axis_index("i")
    k_loc = lax.dynamic_slice_in_dim(K_ext, my * B_LOC, B_LOC, axis=0)
    v_loc = lax.dynamic_slice_in_dim(V_ext, my * B_LOC, B_LOC, axis=0)
    k_t = jnp.transpose(k_loc.astype(jnp.bfloat16), (0, 2, 1, 3))
    v_t = jnp.transpose(v_loc.astype(jnp.bfloat16), (0, 2, 1, 3))
    x_b = x.astype(jnp.bfloat16)
    chunk = jnp.concatenate(
        [Wq.astype(jnp.bfloat16), Wo.T.astype(jnp.bfloat16)], axis=0)

    def body(x_ref, chunk_ref, k_ref, v_ref, out_ref,
             comm, send_sems, recv_sems):
        my_pos = lax.axis_index("i")

        barrier_sem = pltpu.get_barrier_semaphore()
        for j in range(1, N_DEV):
            pl.semaphore_signal(
                barrier_sem, inc=1,
                device_id=(jnp.mod(my_pos + j, N_DEV),),
                device_id_type=pl.DeviceIdType.MESH,
            )
        pl.semaphore_wait(barrier_sem, N_DEV - 1)

        sends = []
        for j in range(1, N_DEV):
            r = pltpu.make_async_remote_copy(
                src_ref=chunk_ref, dst_ref=comm.at[N_DEV - j],
                send_sem=send_sems.at[j - 1],
                recv_sem=recv_sems.at[N_DEV - j - 1],
                device_id=(jnp.mod(my_pos + j, N_DEV),),
                device_id_type=pl.DeviceIdType.MESH,
            )
            r.start()
            sends.append(r)

        qb = lax.broadcasted_iota(jnp.int32, (SQ, SQ), 0) // 64
        kb = lax.broadcasted_iota(jnp.int32, (SQ, SQ), 1) // 64
        mask = (qb == kb) | ((kb % 4) == (qb % 4))

        def compute_chunk(wq_c, woT_c, origin, first=False):
            src = jnp.mod(origin, N_DEV)
            for b in range(B_LOC):
                q_full = jnp.dot(x_ref[b], wq_c,
                                 preferred_element_type=jnp.float32)
                ctx_parts = []
                for h in range(H_LOC):
                    hg = src * H_LOC + h
                    q = q_full[:, h * DH:(h + 1) * DH].astype(jnp.bfloat16)
                    k = k_ref[b, hg]
                    v = v_ref[b, hg]
                    sc = lax.dot_general(
                        q, k, (((1,), (1,)), ((), ())),
                        preferred_element_type=jnp.float32) * 0.125
                    sc = jnp.where(mask, sc, -1e9)
                    m = jnp.max(sc, axis=-1, keepdims=True)
                    w = jnp.exp(sc - m)
                    w = (w / jnp.sum(w, axis=-1, keepdims=True)
                         ).astype(jnp.bfloat16)
                    ctx_parts.append(
                        jnp.dot(w, v, preferred_element_type=jnp.float32))
                ctx = jnp.concatenate(ctx_parts, axis=-1).astype(jnp.bfloat16)
                contrib = lax.dot_general(
                    ctx, woT_c, (((1,), (1,)), ((), ())),
                    preferred_element_type=jnp.float32)
                if first:
                    out_ref[b] = contrib
                else:
                    out_ref[b] = out_ref[b] + contrib

        compute_chunk(chunk_ref[:D_MODEL, :], chunk_ref[D_MODEL:, :],
                      my_pos, first=True)
        for s in _SLOT_ORDER:
            recv = pltpu.make_async_remote_copy(
                src_ref=chunk_ref, dst_ref=comm.at[s],
                send_sem=send_sems.at[0], recv_sem=recv_sems.at[s - 1],
                device_id=(my_pos,), device_id_type=pl.DeviceIdType.MESH,
            )
            recv.wait_recv()
            compute_chunk(comm[s, :D_MODEL, :], comm[s, D_MODEL:, :],
                          my_pos + s)
        for r in sends:
            r.wait_send()

    out_shape = jax.ShapeDtypeStruct((B_LOC, SQ, D_MODEL), jnp.float32)
    return pl.pallas_call(
        body,
        out_shape=out_shape,
        in_specs=[pl.BlockSpec(memory_space=pltpu.VMEM)] * 4,
        out_specs=pl.BlockSpec(memory_space=pltpu.VMEM),
        scratch_shapes=[
            pltpu.VMEM((N_DEV, 2 * D_MODEL, D_CHUNK), jnp.bfloat16),
            pltpu.SemaphoreType.DMA((N_DEV - 1,)),
            pltpu.SemaphoreType.DMA((N_DEV - 1,)),
        ],
        compiler_params=pltpu.CompilerParams(collective_id=0),
    )(x_b, chunk, k_t, v_t)
